# Initial kernel scaffold; baseline (speedup 1.0000x reference)
import functools

import jax
import jax.numpy as jnp
from jax import lax
from jax.experimental import pallas as pl
from jax.experimental.pallas import tpu as pltpu
from jax.experimental.pallas import tpu_sc as plsc

NC = 2
NS = 16
NW = NC * NS
LANES = 16


@functools.lru_cache(maxsize=None)
def _make_kernel(batch, maxlen, embed, vocab):
    rows = batch * maxlen
    assert rows % NW == 0
    rpw = rows // NW
    seqs_per_chunk = 4
    ch = seqs_per_chunk * maxlen
    assert rpw % ch == 0
    nchunk = rpw // ch
    half = embed // 2
    assert embed == 2 * LANES

    mesh = plsc.VectorSubcoreMesh(core_axis_name="c", subcore_axis_name="s")

    @functools.partial(
        pl.kernel,
        out_type=jax.ShapeDtypeStruct((rows, embed), jnp.float32),
        mesh=mesh,
        scratch_types=[
            pltpu.VMEM((ch,), jnp.int32),
            pltpu.VMEM((ch, embed), jnp.float32),
            pltpu.VMEM((maxlen, embed), jnp.float32),
            pltpu.SemaphoreType.DMA,
        ],
    )
    def k(x_hbm, tok_hbm, pos_hbm, out_hbm, idx_v, rows_v, pos_v, sem):
        wid = lax.axis_index("s") * NC + lax.axis_index("c")
        base = wid * rpw
        pltpu.sync_copy(pos_hbm, pos_v)

        def chunk_body(i, carry):
            off = base + i * ch
            pltpu.sync_copy(x_hbm.at[pl.ds(off, ch)], idx_v)
            pltpu.async_copy(tok_hbm.at[idx_v], rows_v, sem).wait()

            def add_t(t, c2):
                p0 = pos_v[t, pl.ds(0, LANES)]
                p1 = pos_v[t, pl.ds(half, LANES)]
                for rr in range(seqs_per_chunk):
                    r = rr * maxlen + t
                    rows_v[r, pl.ds(0, LANES)] = rows_v[r, pl.ds(0, LANES)] + p0
                    rows_v[r, pl.ds(half, LANES)] = (
                        rows_v[r, pl.ds(half, LANES)] + p1
                    )
                return c2

            lax.fori_loop(0, maxlen, add_t, 0)
            pltpu.sync_copy(rows_v, out_hbm.at[pl.ds(off, ch)])
            return carry

        lax.fori_loop(0, nchunk, chunk_body, 0)

    return k


def kernel(x, token_table, pos_table):
    batch, maxlen = x.shape
    vocab, embed = token_table.shape
    xf = x.reshape(-1).astype(jnp.int32)
    out = _make_kernel(batch, maxlen, embed, vocab)(
        xf, token_table, pos_table
    )
    return out.reshape(batch, maxlen, embed)


# speedup vs baseline: 1.3953x; 1.3953x over previous
import functools

import jax
import jax.numpy as jnp
from jax import lax
from jax.experimental import pallas as pl
from jax.experimental.pallas import tpu as pltpu
from jax.experimental.pallas import tpu_sc as plsc

NC = 2
NS = 16
NW = NC * NS
LANES = 16


@functools.lru_cache(maxsize=None)
def _make_kernel(batch, maxlen, embed, vocab):
    rows = batch * maxlen
    assert rows % NW == 0
    rpw = rows // NW
    seqs_per_chunk = 4
    ch = seqs_per_chunk * maxlen
    assert rpw % ch == 0
    nchunk = rpw // ch
    half = embed // 2
    assert embed == 2 * LANES

    mesh = plsc.VectorSubcoreMesh(core_axis_name="c", subcore_axis_name="s")

    @functools.partial(
        pl.kernel,
        out_type=jax.ShapeDtypeStruct((rows, embed), jnp.float32),
        mesh=mesh,
        scratch_types=[
            pltpu.VMEM((ch,), jnp.int32),
            pltpu.VMEM((ch, embed), jnp.float32),
            pltpu.VMEM((maxlen, embed), jnp.float32),
            pltpu.SemaphoreType.DMA,
        ],
        compiler_params=pltpu.CompilerParams(use_tc_tiling_on_sc=False),
    )
    def k(x_hbm, tok_hbm, pos_hbm, out_hbm, idx_v, rows_v, pos_v, sem):
        wid = lax.axis_index("s") * NC + lax.axis_index("c")
        base = wid * rpw
        pltpu.sync_copy(pos_hbm, pos_v)

        def chunk_body(i, carry):
            off = base + i * ch
            pltpu.sync_copy(x_hbm.at[pl.ds(off, ch)], idx_v)
            pltpu.async_copy(tok_hbm.at[idx_v], rows_v, sem).wait()

            def add_t(t, c2):
                p0 = pos_v[t, pl.ds(0, LANES)]
                p1 = pos_v[t, pl.ds(half, LANES)]
                for rr in range(seqs_per_chunk):
                    r = rr * maxlen + t
                    rows_v[r, pl.ds(0, LANES)] = rows_v[r, pl.ds(0, LANES)] + p0
                    rows_v[r, pl.ds(half, LANES)] = (
                        rows_v[r, pl.ds(half, LANES)] + p1
                    )
                return c2

            lax.fori_loop(0, maxlen, add_t, 0)
            pltpu.sync_copy(rows_v, out_hbm.at[pl.ds(off, ch)])
            return carry

        lax.fori_loop(0, nchunk, chunk_body, 0)

    return k


def kernel(x, token_table, pos_table):
    batch, maxlen = x.shape
    vocab, embed = token_table.shape
    xf = x.reshape(-1).astype(jnp.int32)
    out = _make_kernel(batch, maxlen, embed, vocab)(
        xf, token_table, pos_table
    )
    return out.reshape(batch, maxlen, embed)


# speedup vs baseline: 1.4910x; 1.0686x over previous
import functools

import jax
import jax.numpy as jnp
from jax import lax
from jax.experimental import pallas as pl
from jax.experimental.pallas import tpu as pltpu
from jax.experimental.pallas import tpu_sc as plsc

NC = 2
NS = 16
NW = NC * NS
LANES = 16
SEQS_PER_CHUNK = 4


@functools.lru_cache(maxsize=None)
def _make_kernel(batch, maxlen, embed, vocab):
    rows = batch * maxlen
    assert rows % NW == 0
    rpw = rows // NW
    ch = SEQS_PER_CHUNK * maxlen
    assert rpw % (2 * ch) == 0
    nchunk = rpw // ch
    half = embed // 2
    assert embed == 2 * LANES

    mesh = plsc.VectorSubcoreMesh(core_axis_name="c", subcore_axis_name="s")

    @functools.partial(
        pl.kernel,
        out_type=jax.ShapeDtypeStruct((rows, embed), jnp.float32),
        mesh=mesh,
        scratch_types=[
            pltpu.VMEM((ch,), jnp.int32),
            pltpu.VMEM((ch,), jnp.int32),
            pltpu.VMEM((ch, embed), jnp.float32),
            pltpu.VMEM((ch, embed), jnp.float32),
            pltpu.VMEM((maxlen, embed), jnp.float32),
            pltpu.SemaphoreType.DMA,
            pltpu.SemaphoreType.DMA,
            pltpu.SemaphoreType.DMA,
            pltpu.SemaphoreType.DMA,
            pltpu.SemaphoreType.DMA,
            pltpu.SemaphoreType.DMA,
        ],
        compiler_params=pltpu.CompilerParams(use_tc_tiling_on_sc=False),
    )
    def k(x_hbm, tok_hbm, pos_hbm, out_hbm,
          ib0, ib1, rows0, rows1, pos_v,
          si0, si1, sg0, sg1, so0, so1):
        wid = lax.axis_index("s") * NC + lax.axis_index("c")
        base = wid * rpw
        ib = (ib0, ib1)
        bufs = (rows0, rows1)
        si = (si0, si1)
        sg = (sg0, sg1)
        so = (so0, so1)

        pltpu.sync_copy(pos_hbm, pos_v)
        pltpu.sync_copy(x_hbm.at[wid, 0], ib0)
        pltpu.async_copy(tok_hbm.at[ib0], rows0, sg0)
        pltpu.async_copy(x_hbm.at[wid, 1], ib1, si1)

        def pair_body(gi, carry):
            for b in range(2):
                i = 2 * gi + b
                rb, rnb = bufs[b], bufs[1 - b]

                @pl.when(i >= 1)
                def _wait_prev_out():
                    pltpu.make_async_copy(
                        rnb, out_hbm.at[pl.ds(base + (i - 1) * ch, ch)],
                        so[1 - b]).wait()

                @pl.when(i + 1 < nchunk)
                def _fire_next_gather():
                    pltpu.make_async_copy(
                        x_hbm.at[wid, i + 1], ib[1 - b], si[1 - b]).wait()
                    pltpu.async_copy(tok_hbm.at[ib[1 - b]], rnb, sg[1 - b])

                pltpu.make_async_copy(tok_hbm.at[ib[b]], rb, sg[b]).wait()

                @pl.when(i + 2 < nchunk)
                def _fetch_next_idx():
                    pltpu.async_copy(x_hbm.at[wid, i + 2], ib[b], si[b])

                @plsc.parallel_loop(0, maxlen, unroll=2)
                def _add_t(t):
                    p0 = pos_v[t, pl.ds(0, LANES)]
                    p1 = pos_v[t, pl.ds(half, LANES)]
                    for rr in range(SEQS_PER_CHUNK):
                        r = rr * maxlen + t
                        rb[r, pl.ds(0, LANES)] = rb[r, pl.ds(0, LANES)] + p0
                        rb[r, pl.ds(half, LANES)] = (
                            rb[r, pl.ds(half, LANES)] + p1
                        )

                pltpu.async_copy(
                    rb, out_hbm.at[pl.ds(base + i * ch, ch)], so[b])
            return carry

        lax.fori_loop(0, nchunk // 2, pair_body, 0)
        pltpu.make_async_copy(
            bufs[1], out_hbm.at[pl.ds(base + (nchunk - 1) * ch, ch)],
            so[1]).wait()

    return k


def kernel(x, token_table, pos_table):
    batch, maxlen = x.shape
    vocab, embed = token_table.shape
    rows = batch * maxlen
    rpw = rows // NW
    ch = SEQS_PER_CHUNK * maxlen
    xf = x.reshape(NW, rpw // ch, ch).astype(jnp.int32)
    out = _make_kernel(batch, maxlen, embed, vocab)(
        xf, token_table, pos_table
    )
    return out.reshape(batch, maxlen, embed)
